# baseline (speedup 1.0000x reference)
import functools

import jax
import jax.numpy as jnp
from jax.experimental import pallas as pl
from jax.experimental.pallas import tpu as pltpu

_N = 200
_ROWS = 1024
_CHUNK = 8
_KB = 40


def _listmle_kernel(pred_ref, true_ref, out_ref, e_ref, t_ref):
    predT = pred_ref[...].T
    r = predT.shape[1]
    t_ref[...] = true_ref[...].T
    m = jnp.max(predT, axis=0, keepdims=True)
    e_ref[...] = jnp.exp(predT - m)
    nchunks = _N // _CHUNK
    cpb = _KB // _CHUNK

    logsum = jnp.zeros((1, r), jnp.float32)
    for kb in range(_N // _KB):
        kbase = kb * _KB
        trb = t_ref[pl.ds(kbase, _KB), :]
        kidx = kbase + jax.lax.broadcasted_iota(jnp.int32, (_KB, 1), 0)

        a0 = jnp.zeros((_KB, r), jnp.float32)
        a1 = a0
        for jj in range(nchunks):
            base = jj * _CHUNK
            tch = t_ref[pl.ds(base, _CHUNK), :]
            ech = e_ref[pl.ds(base, _CHUNK), :]
            for s in range(_CHUNK):
                tj = tch[s:s + 1, :]
                ej = ech[s:s + 1, :]
                j = base + s
                if j < kbase:
                    mask = trb > tj
                elif j >= kbase + _KB:
                    mask = trb >= tj
                else:
                    mask = (trb > tj) | ((trb >= tj) & (kidx <= j))
                contrib = jnp.where(mask, ej, 0.0)
                if s % 2 == 0:
                    a0 = a0 + contrib
                else:
                    a1 = a1 + contrib
        t = a0 + a1
        logsum = logsum + jnp.sum(jnp.log(t), axis=0, keepdims=True)

    loss_rows = (logsum
                 + _N * m
                 - jnp.sum(predT, axis=0, keepdims=True))
    out_ref[pl.program_id(0), 0] = jnp.sum(loss_rows)


@jax.jit
def kernel(pred, true):
    grid = 4
    r = _ROWS // grid
    partials = pl.pallas_call(
        _listmle_kernel,
        grid=(grid,),
        in_specs=[
            pl.BlockSpec((r, _N), lambda i: (i, 0)),
            pl.BlockSpec((r, _N), lambda i: (i, 0)),
        ],
        out_specs=pl.BlockSpec((grid, 1), lambda i: (0, 0),
                               memory_space=pltpu.SMEM),
        out_shape=jax.ShapeDtypeStruct((grid, 1), jnp.float32),
        scratch_shapes=[pltpu.VMEM((_N, r), jnp.float32),
                        pltpu.VMEM((_N, r), jnp.float32)],
        compiler_params=pltpu.CompilerParams(
            dimension_semantics=("parallel",),
        ),
    )(pred, true)
    return jnp.sum(partials) / _ROWS


# speedup vs baseline: 1.1839x; 1.1839x over previous
import functools

import jax
import jax.numpy as jnp
from jax.experimental import pallas as pl
from jax.experimental.pallas import tpu as pltpu

_N = 200
_ROWS = 1024
_CHUNK = 8
_KB = 40


def _listmle_kernel(pred_ref, true_ref, out_ref, e_ref):
    predT = pred_ref[...]
    r = predT.shape[1]
    m = jnp.max(predT, axis=0, keepdims=True)
    e_ref[...] = jnp.exp(predT - m)
    nchunks = _N // _CHUNK
    cpb = _KB // _CHUNK

    logsum = jnp.zeros((1, r), jnp.float32)
    for kb in range(_N // _KB):
        kbase = kb * _KB
        trb = true_ref[pl.ds(kbase, _KB), :]
        kidx = kbase + jax.lax.broadcasted_iota(jnp.int32, (_KB, 1), 0)

        a0 = jnp.zeros((_KB, r), jnp.float32)
        a1 = a0
        for jj in range(nchunks):
            base = jj * _CHUNK
            tch = true_ref[pl.ds(base, _CHUNK), :]
            ech = e_ref[pl.ds(base, _CHUNK), :]
            for s in range(_CHUNK):
                tj = tch[s:s + 1, :]
                ej = ech[s:s + 1, :]
                j = base + s
                if j < kbase:
                    mask = trb > tj
                elif j >= kbase + _KB:
                    mask = trb >= tj
                else:
                    mask = (trb > tj) | ((trb >= tj) & (kidx <= j))
                contrib = jnp.where(mask, ej, 0.0)
                if s % 2 == 0:
                    a0 = a0 + contrib
                else:
                    a1 = a1 + contrib
        t = a0 + a1
        logsum = logsum + jnp.sum(jnp.log(t), axis=0, keepdims=True)

    loss_rows = (logsum
                 + _N * m
                 - jnp.sum(predT, axis=0, keepdims=True))
    out_ref[pl.program_id(0), 0] = jnp.sum(loss_rows)


@jax.jit
def kernel(pred, true):
    grid = 8
    r = _ROWS // grid
    predT = pred.T
    trueT = true.T
    partials = pl.pallas_call(
        _listmle_kernel,
        grid=(grid,),
        in_specs=[
            pl.BlockSpec((_N, r), lambda i: (0, i)),
            pl.BlockSpec((_N, r), lambda i: (0, i)),
        ],
        out_specs=pl.BlockSpec((grid, 1), lambda i: (0, 0),
                               memory_space=pltpu.SMEM),
        out_shape=jax.ShapeDtypeStruct((grid, 1), jnp.float32),
        scratch_shapes=[pltpu.VMEM((_N, r), jnp.float32)],
        compiler_params=pltpu.CompilerParams(
            dimension_semantics=("parallel",),
        ),
    )(predT, trueT)
    return jnp.sum(partials) / _ROWS


# speedup vs baseline: 1.1940x; 1.0085x over previous
import functools

import jax
import jax.numpy as jnp
from jax.experimental import pallas as pl
from jax.experimental.pallas import tpu as pltpu

_N = 200
_ROWS = 1024
_CHUNK = 8
_NV = _N // _CHUNK


def _listmle_kernel(pred_ref, true_ref, out_ref, e_ref):
    predT = pred_ref[...]
    r = predT.shape[1]
    m = jnp.max(predT, axis=0, keepdims=True)
    e_ref[...] = jnp.exp(predT - m)
    siota = jax.lax.broadcasted_iota(jnp.int32, (_CHUNK, 1), 0)

    acc = [jnp.zeros((_CHUNK, r), jnp.float32) for _ in range(_NV)]
    trv = [true_ref[pl.ds(kv * _CHUNK, _CHUNK), :] for kv in range(_NV)]
    for jj in range(_NV):
        base = jj * _CHUNK
        tch = true_ref[pl.ds(base, _CHUNK), :]
        ech = e_ref[pl.ds(base, _CHUNK), :]
        for s in range(_CHUNK):
            j = base + s
            tj = tch[s:s + 1, :]
            ej = ech[s:s + 1, :]
            for kv in range(_NV):
                if kv < jj:
                    mask = trv[kv] >= tj
                elif kv > jj:
                    mask = trv[kv] > tj
                else:
                    kmask = siota <= s
                    mask = (trv[kv] > tj) | ((trv[kv] >= tj) & kmask)
                acc[kv] = acc[kv] + jnp.where(mask, ej, 0.0)

    logsum = jnp.zeros((1, r), jnp.float32)
    for kv in range(_NV):
        logsum = logsum + jnp.sum(jnp.log(acc[kv]), axis=0, keepdims=True)
    loss_rows = (logsum
                 + _N * m
                 - jnp.sum(predT, axis=0, keepdims=True))
    out_ref[pl.program_id(0), 0] = jnp.sum(loss_rows)


@jax.jit
def kernel(pred, true):
    grid = 8
    r = _ROWS // grid
    predT = pred.T
    trueT = true.T
    partials = pl.pallas_call(
        _listmle_kernel,
        grid=(grid,),
        in_specs=[
            pl.BlockSpec((_N, r), lambda i: (0, i)),
            pl.BlockSpec((_N, r), lambda i: (0, i)),
        ],
        out_specs=pl.BlockSpec((grid, 1), lambda i: (0, 0),
                               memory_space=pltpu.SMEM),
        out_shape=jax.ShapeDtypeStruct((grid, 1), jnp.float32),
        scratch_shapes=[pltpu.VMEM((_N, r), jnp.float32)],
        compiler_params=pltpu.CompilerParams(
            dimension_semantics=("parallel",),
        ),
    )(predT, trueT)
    return jnp.sum(partials) / _ROWS


# speedup vs baseline: 1.2217x; 1.0232x over previous
import functools

import jax
import jax.numpy as jnp
from jax.experimental import pallas as pl
from jax.experimental.pallas import tpu as pltpu

_N = 200
_ROWS = 1024
_CHUNK = 8
_NV = _N // _CHUNK


def _listmle_kernel(pred_ref, true_ref, out_ref, e_ref):
    predT = pred_ref[...]
    r = predT.shape[1]
    m = jnp.max(predT, axis=0, keepdims=True)
    e_ref[...] = jnp.exp(predT - m)
    siota = jax.lax.broadcasted_iota(jnp.int32, (_CHUNK, 1), 0)

    acc = [jnp.zeros((_CHUNK, r), jnp.float32) for _ in range(_NV)]
    trv = [true_ref[pl.ds(kv * _CHUNK, _CHUNK), :] for kv in range(_NV)]
    for jj in range(_NV):
        base = jj * _CHUNK
        tch = true_ref[pl.ds(base, _CHUNK), :]
        ech = e_ref[pl.ds(base, _CHUNK), :]
        for s in range(_CHUNK):
            j = base + s
            tj = tch[s:s + 1, :]
            ej = ech[s:s + 1, :]
            for kv in range(_NV):
                if kv < jj:
                    mask = trv[kv] >= tj
                elif kv > jj:
                    mask = trv[kv] > tj
                else:
                    kmask = siota <= s
                    mask = (trv[kv] > tj) | ((trv[kv] >= tj) & kmask)
                acc[kv] = acc[kv] + jnp.where(mask, ej, 0.0)

    logsum = jnp.zeros((1, r), jnp.float32)
    for kv in range(_NV):
        logsum = logsum + jnp.sum(jnp.log(acc[kv]), axis=0, keepdims=True)
    loss_rows = (logsum
                 + _N * m
                 - jnp.sum(predT, axis=0, keepdims=True))
    out_ref[pl.program_id(0), 0] = jnp.sum(loss_rows)


@jax.jit
def kernel(pred, true):
    grid = 4
    r = _ROWS // grid
    predT = pred.T
    trueT = true.T
    partials = pl.pallas_call(
        _listmle_kernel,
        grid=(grid,),
        in_specs=[
            pl.BlockSpec((_N, r), lambda i: (0, i)),
            pl.BlockSpec((_N, r), lambda i: (0, i)),
        ],
        out_specs=pl.BlockSpec((grid, 1), lambda i: (0, 0),
                               memory_space=pltpu.SMEM),
        out_shape=jax.ShapeDtypeStruct((grid, 1), jnp.float32),
        scratch_shapes=[pltpu.VMEM((_N, r), jnp.float32)],
        compiler_params=pltpu.CompilerParams(
            dimension_semantics=("parallel",),
        ),
    )(predT, trueT)
    return jnp.sum(partials) / _ROWS
